# baseline (speedup 1.0000x reference)
import jax
import jax.numpy as jnp
import numpy as np
from jax import lax
from jax.experimental import pallas as pl
from jax.experimental.pallas import tpu as pltpu
from jax.experimental.pallas import tpu_sc as plsc

N = 10000
DI = 128
DO = 64
R = 3
E = 320000
NC = 2
NS = 16

_ST = [s * 632 for s in range(NS - 1)] + [632 * (NS - 1)]
_LN = [632] * (NS - 1) + [N - 632 * (NS - 1)]

_RPS = N // NS
_EW_A = E // (NC * NS)
_KA = 2000
_EW_C = E // NS
_KC = 200
_NCH = _EW_C // _KC
_NPAIR = (_NCH - 2) // 2

_BR = 1000
_NB = N // _BR

_PERM_NP = np.asarray(
    jax.random.permutation(jax.random.key(1), N)
).astype(np.int32).reshape(NS, _RPS)


def _vec_mesh():
    return plsc.VectorSubcoreMesh(core_axis_name="core", subcore_axis_name="subcore")


_SC_PARAMS = pltpu.CompilerParams(use_tc_tiling_on_sc=False)


def _deg_perm(dst, x, perm2d, z1):
    kfn = pl.kernel(
        _deg_perm_body,
        out_type=(
            jax.ShapeDtypeStruct((NC, R, N), jnp.float32),
            jax.ShapeDtypeStruct((N, DI), jnp.float32),
        ),
        mesh=_vec_mesh(),
        scratch_types=[
            pltpu.VMEM_SHARED((N,), jnp.float32),
            pltpu.VMEM_SHARED((N,), jnp.float32),
            pltpu.VMEM_SHARED((N,), jnp.float32),
            pltpu.VMEM((_KA,), jnp.float32),
            pltpu.VMEM((_KA,), jnp.int32),
            pltpu.VMEM((_RPS,), jnp.int32),
            pltpu.VMEM((_RPS, DI), jnp.float32),
        ],
        compiler_params=_SC_PARAMS,
    )
    return kfn(dst, x, perm2d, z1)


def _deg_perm_body(dst_hbm, x_hbm, perm_hbm, z1_hbm, degp_hbm, xperm_hbm,
                   d0_sh, d1_sh, d2_sh, ones_v, idx_v, perm_v, xrow_v):
    c = lax.axis_index("core")
    s = lax.axis_index("subcore")
    degs = [d0_sh, d1_sh, d2_sh]

    @pl.loop(0, _KA, step=16)
    def _(i):
        ones_v[pl.ds(i, 16)] = jnp.full((16,), 1.0, jnp.float32)

    for i in range(NS):
        @pl.when(s == i)
        def _():
            for r in range(R):
                pltpu.sync_copy(z1_hbm.at[pl.ds(_ST[i], _LN[i])],
                                degs[r].at[pl.ds(_ST[i], _LN[i])])
    plsc.subcore_barrier()

    @pl.when(c == 0)
    def _():
        pltpu.sync_copy(perm_hbm.at[s], perm_v)
        pltpu.sync_copy(x_hbm.at[perm_v], xrow_v)
        pltpu.sync_copy(xrow_v, xperm_hbm.at[pl.ds(s * _RPS, _RPS)])

    base0 = (c * NS + s) * _EW_A
    for r in range(R):
        @pl.loop(0, _EW_A, step=_KA)
        def _(j):
            pltpu.sync_copy(dst_hbm.at[r, pl.ds(base0 + j, _KA)], idx_v)
            pltpu.sync_copy(ones_v, degs[r].at[idx_v], add=True)
    plsc.subcore_barrier()

    for i in range(NS):
        @pl.when(s == i)
        def _():
            for r in range(R):
                pltpu.sync_copy(degs[r].at[pl.ds(_ST[i], _LN[i])],
                                degp_hbm.at[c, r, pl.ds(_ST[i], _LN[i])])


def _tables_body(x_ref, xp_ref, w_ref, degp_ref, t_ref, dinv_ref):
    deg = degp_ref[0, 0, :, 0] + degp_ref[1, 0, :, 0] + 1.0
    dv = lax.rsqrt(deg).reshape(_BR, 1)
    w = w_ref[0]
    xw = jnp.dot(x_ref[...], w, preferred_element_type=jnp.float32,
                 precision=lax.Precision.HIGHEST)
    xwp = jnp.dot(xp_ref[...], w, preferred_element_type=jnp.float32,
                  precision=lax.Precision.HIGHEST)
    t_ref[0, 0] = dv * xw
    t_ref[1, 0] = dv * xwp
    dinv_ref[0] = dv


def _tables(x, xperm, Ws, degp):
    return pl.pallas_call(
        _tables_body,
        grid=(R, _NB),
        in_specs=[
            pl.BlockSpec((_BR, DI), lambda r, b: (b, 0)),
            pl.BlockSpec((_BR, DI), lambda r, b: (b, 0)),
            pl.BlockSpec((1, DI, DO), lambda r, b: (r, 0, 0)),
            pl.BlockSpec((NC, 1, _BR, 1), lambda r, b: (0, r, b, 0)),
        ],
        out_specs=[
            pl.BlockSpec((NC, 1, _BR, DO), lambda r, b: (0, r, b, 0)),
            pl.BlockSpec((1, _BR, 1), lambda r, b: (r, b, 0)),
        ],
        out_shape=[
            jax.ShapeDtypeStruct((NC, R, N, DO), jnp.float32),
            jax.ShapeDtypeStruct((R, N, 1), jnp.float32),
        ],
    )(x, xperm, Ws, degp)


def _edge(src, dst, t, z2):
    kfn = pl.kernel(
        _edge_body,
        out_type=jax.ShapeDtypeStruct((NC, R, N, DO), jnp.float32),
        mesh=_vec_mesh(),
        scratch_types=[
            pltpu.VMEM_SHARED((N, DO), jnp.float32),
            pltpu.VMEM_SHARED((N, DO), jnp.float32),
            pltpu.VMEM((_KC,), jnp.int32),
            pltpu.VMEM((_KC,), jnp.int32),
            pltpu.VMEM((_KC,), jnp.int32),
            pltpu.VMEM((_KC,), jnp.int32),
            pltpu.VMEM((_KC, DO), jnp.float32),
            pltpu.VMEM((_KC, DO), jnp.float32),
            pltpu.SemaphoreType.DMA,
            pltpu.SemaphoreType.DMA,
        ],
        compiler_params=_SC_PARAMS,
    )
    return kfn(src, dst, t, z2)


def _edge_body(src_hbm, dst_hbm, t_hbm, z2_hbm, acc_hbm,
               tbl_sh, acc_sh, srcA, srcB, dstA, dstB, rowA, rowB, semA, semB):
    c = lax.axis_index("core")
    s = lax.axis_index("subcore")
    rs = pl.ds(s * _RPS, _RPS)
    base0 = s * _EW_C
    for r in range(R):
        pltpu.sync_copy(z2_hbm.at[rs], acc_sh.at[rs])
        pltpu.sync_copy(t_hbm.at[c, r].at[rs], tbl_sh.at[rs])
        plsc.subcore_barrier()

        tcr = tbl_sh
        pltpu.sync_copy(src_hbm.at[r, pl.ds(base0, _KC)], srcA)
        pltpu.sync_copy(dst_hbm.at[r, pl.ds(base0, _KC)], dstA)
        pltpu.async_copy(tcr.at[srcA], rowA, semA)

        @pl.loop(0, _NPAIR * 2 * _KC, step=2 * _KC)
        def _(j):
            pltpu.sync_copy(src_hbm.at[r, pl.ds(base0 + j + _KC, _KC)], srcB)
            pltpu.sync_copy(dst_hbm.at[r, pl.ds(base0 + j + _KC, _KC)], dstB)
            pltpu.async_copy(tcr.at[srcB], rowB, semB)
            pltpu.make_async_copy(tcr.at[srcA], rowA, semA).wait()
            pltpu.sync_copy(rowA, acc_sh.at[dstA], add=True)

            nj = j + 2 * _KC
            pltpu.sync_copy(src_hbm.at[r, pl.ds(base0 + nj, _KC)], srcA)
            pltpu.sync_copy(dst_hbm.at[r, pl.ds(base0 + nj, _KC)], dstA)
            pltpu.async_copy(tcr.at[srcA], rowA, semA)
            pltpu.make_async_copy(tcr.at[srcB], rowB, semB).wait()
            pltpu.sync_copy(rowB, acc_sh.at[dstB], add=True)

        pltpu.make_async_copy(tcr.at[srcA], rowA, semA).wait()
        pltpu.sync_copy(rowA, acc_sh.at[dstA], add=True)
        last = base0 + (_NCH - 1) * _KC
        pltpu.sync_copy(src_hbm.at[r, pl.ds(last, _KC)], srcB)
        pltpu.sync_copy(dst_hbm.at[r, pl.ds(last, _KC)], dstB)
        pltpu.sync_copy(tcr.at[srcB], rowB)
        pltpu.sync_copy(rowB, acc_sh.at[dstB], add=True)

        plsc.subcore_barrier()
        pltpu.sync_copy(acc_sh.at[rs], acc_hbm.at[c, r].at[rs])
        plsc.subcore_barrier()


def _final_body(acc_ref, t_ref, dinv_ref, b_ref, pos_ref, neg_ref, sum_ref):
    dv = dinv_ref[0]
    bias = b_ref[0, 0]
    p = jnp.maximum(dv * (acc_ref[0, 0] + t_ref[0, 0]) + bias, 0.0)
    n = jnp.maximum(dv * (acc_ref[1, 0] + t_ref[1, 0]) + bias, 0.0)
    pos_ref[0] = p
    neg_ref[0] = n
    part = (jnp.sum(p, axis=0) * (1.0 / N)).reshape(1, 1, DO)
    b = pl.program_id(1)

    @pl.when(b == 0)
    def _():
        sum_ref[...] = part

    @pl.when(b > 0)
    def _():
        sum_ref[...] += part


def _final(acc, t, dinv, bs3):
    return pl.pallas_call(
        _final_body,
        grid=(R, _NB),
        in_specs=[
            pl.BlockSpec((NC, 1, _BR, DO), lambda r, b: (0, r, b, 0)),
            pl.BlockSpec((NC, 1, _BR, DO), lambda r, b: (0, r, b, 0)),
            pl.BlockSpec((1, _BR, 1), lambda r, b: (r, b, 0)),
            pl.BlockSpec((1, 1, DO), lambda r, b: (r, 0, 0)),
        ],
        out_specs=[
            pl.BlockSpec((1, _BR, DO), lambda r, b: (r, b, 0)),
            pl.BlockSpec((1, _BR, DO), lambda r, b: (r, b, 0)),
            pl.BlockSpec((1, 1, DO), lambda r, b: (r, 0, 0)),
        ],
        out_shape=[
            jax.ShapeDtypeStruct((R, N, DO), jnp.float32),
            jax.ShapeDtypeStruct((R, N, DO), jnp.float32),
            jax.ShapeDtypeStruct((R, 1, DO), jnp.float32),
        ],
    )(acc, t, dinv, bs3)


def kernel(x, edge_indices, Ws, bs):
    src = edge_indices[:, 0, :].astype(jnp.int32)
    dst = edge_indices[:, 1, :].astype(jnp.int32)
    perm = jax.random.permutation(jax.random.key(1), N)
    perm2d = perm.astype(jnp.int32).reshape(NS, _RPS)
    z1 = jnp.zeros((N,), jnp.float32)
    z2 = jnp.zeros((N, DO), jnp.float32)

    degp, xperm = _deg_perm(dst, x, perm2d, z1)
    t, dinv = _tables(x, xperm, Ws, degp.reshape(NC, R, N, 1))
    acc = _edge(src, dst, t, z2)
    pos, neg, summ = _final(acc, t, dinv, bs.reshape(R, 1, DO))
    return pos, neg, summ


# speedup vs baseline: 1.1349x; 1.1349x over previous
import jax
import jax.numpy as jnp
import numpy as np
from jax import lax
from jax.experimental import pallas as pl
from jax.experimental.pallas import tpu as pltpu
from jax.experimental.pallas import tpu_sc as plsc

N = 10000
DI = 128
DO = 64
R = 3
E = 320000
NC = 2
NS = 16

_ST = [s * 632 for s in range(NS - 1)] + [632 * (NS - 1)]
_LN = [632] * (NS - 1) + [N - 632 * (NS - 1)]

_RPS = N // NS
_EW_A = E // (NC * NS)
_KA = 2000
_EW_C = E // NS
_KC = 400
_NCH = _EW_C // _KC
_NPAIR = (_NCH - 2) // 2

_BR = 1000
_NB = N // _BR

_PERM_NP = np.asarray(
    jax.random.permutation(jax.random.key(1), N)
).astype(np.int32).reshape(NS, _RPS)


def _vec_mesh():
    return plsc.VectorSubcoreMesh(core_axis_name="core", subcore_axis_name="subcore")


_SC_PARAMS = pltpu.CompilerParams(use_tc_tiling_on_sc=False)


def _deg_perm(dst, x, perm2d, z1):
    kfn = pl.kernel(
        _deg_perm_body,
        out_type=(
            jax.ShapeDtypeStruct((NC, R, N), jnp.float32),
            jax.ShapeDtypeStruct((N, DI), jnp.float32),
        ),
        mesh=_vec_mesh(),
        scratch_types=[
            pltpu.VMEM_SHARED((N,), jnp.float32),
            pltpu.VMEM_SHARED((N,), jnp.float32),
            pltpu.VMEM_SHARED((N,), jnp.float32),
            pltpu.VMEM((_KA,), jnp.float32),
            pltpu.VMEM((_KA,), jnp.int32),
            pltpu.VMEM((_RPS,), jnp.int32),
            pltpu.VMEM((_RPS, DI), jnp.float32),
        ],
        compiler_params=_SC_PARAMS,
    )
    return kfn(dst, x, perm2d, z1)


def _deg_perm_body(dst_hbm, x_hbm, perm_hbm, z1_hbm, degp_hbm, xperm_hbm,
                   d0_sh, d1_sh, d2_sh, ones_v, idx_v, perm_v, xrow_v):
    c = lax.axis_index("core")
    s = lax.axis_index("subcore")
    degs = [d0_sh, d1_sh, d2_sh]

    @pl.loop(0, _KA, step=16)
    def _(i):
        ones_v[pl.ds(i, 16)] = jnp.full((16,), 1.0, jnp.float32)

    for i in range(NS):
        @pl.when(s == i)
        def _():
            for r in range(R):
                pltpu.sync_copy(z1_hbm.at[pl.ds(_ST[i], _LN[i])],
                                degs[r].at[pl.ds(_ST[i], _LN[i])])
    plsc.subcore_barrier()

    @pl.when(c == 0)
    def _():
        pltpu.sync_copy(perm_hbm.at[s], perm_v)
        pltpu.sync_copy(x_hbm.at[perm_v], xrow_v)
        pltpu.sync_copy(xrow_v, xperm_hbm.at[pl.ds(s * _RPS, _RPS)])

    base0 = (c * NS + s) * _EW_A
    for r in range(R):
        @pl.loop(0, _EW_A, step=_KA)
        def _(j):
            pltpu.sync_copy(dst_hbm.at[r, pl.ds(base0 + j, _KA)], idx_v)
            pltpu.sync_copy(ones_v, degs[r].at[idx_v], add=True)
    plsc.subcore_barrier()

    for i in range(NS):
        @pl.when(s == i)
        def _():
            for r in range(R):
                pltpu.sync_copy(degs[r].at[pl.ds(_ST[i], _LN[i])],
                                degp_hbm.at[c, r, pl.ds(_ST[i], _LN[i])])


def _tables_body(x_ref, xp_ref, w_ref, degp_ref, t_ref, dinv_ref):
    deg = degp_ref[0, 0, :, 0] + degp_ref[1, 0, :, 0] + 1.0
    dv = lax.rsqrt(deg).reshape(_BR, 1)
    w = w_ref[0]
    xw = jnp.dot(x_ref[...], w, preferred_element_type=jnp.float32,
                 precision=lax.Precision.HIGHEST)
    xwp = jnp.dot(xp_ref[...], w, preferred_element_type=jnp.float32,
                  precision=lax.Precision.HIGHEST)
    t_ref[0, 0] = dv * xw
    t_ref[1, 0] = dv * xwp
    dinv_ref[0] = dv


def _tables(x, xperm, Ws, degp):
    return pl.pallas_call(
        _tables_body,
        grid=(R, _NB),
        in_specs=[
            pl.BlockSpec((_BR, DI), lambda r, b: (b, 0)),
            pl.BlockSpec((_BR, DI), lambda r, b: (b, 0)),
            pl.BlockSpec((1, DI, DO), lambda r, b: (r, 0, 0)),
            pl.BlockSpec((NC, 1, _BR, 1), lambda r, b: (0, r, b, 0)),
        ],
        out_specs=[
            pl.BlockSpec((NC, 1, _BR, DO), lambda r, b: (0, r, b, 0)),
            pl.BlockSpec((1, _BR, 1), lambda r, b: (r, b, 0)),
        ],
        out_shape=[
            jax.ShapeDtypeStruct((NC, R, N, DO), jnp.float32),
            jax.ShapeDtypeStruct((R, N, 1), jnp.float32),
        ],
    )(x, xperm, Ws, degp)


def _edge(src, dst, t, z2):
    kfn = pl.kernel(
        _edge_body,
        out_type=jax.ShapeDtypeStruct((NC, R, N, DO), jnp.float32),
        mesh=_vec_mesh(),
        scratch_types=[
            pltpu.VMEM_SHARED((N, DO), jnp.float32),
            pltpu.VMEM_SHARED((N, DO), jnp.float32),
            pltpu.VMEM((_KC,), jnp.int32),
            pltpu.VMEM((_KC,), jnp.int32),
            pltpu.VMEM((_KC, DO), jnp.float32),
        ],
        compiler_params=_SC_PARAMS,
    )
    return kfn(src, dst, t, z2)


def _edge_body(src_hbm, dst_hbm, t_hbm, z2_hbm, acc_hbm,
               tbl_sh, acc_sh, srcA, dstA, rowA):
    c = lax.axis_index("core")
    s = lax.axis_index("subcore")
    rs = pl.ds(s * _RPS, _RPS)
    base0 = s * _EW_C
    for r in range(R):
        pltpu.sync_copy(z2_hbm.at[rs], acc_sh.at[rs])
        pltpu.sync_copy(t_hbm.at[c, r].at[rs], tbl_sh.at[rs])
        plsc.subcore_barrier()

        @pl.loop(0, _NCH * _KC, step=_KC)
        def _(j):
            pltpu.sync_copy(src_hbm.at[r, pl.ds(base0 + j, _KC)], srcA)
            pltpu.sync_copy(dst_hbm.at[r, pl.ds(base0 + j, _KC)], dstA)
            pltpu.sync_copy(tbl_sh.at[srcA], rowA)
            pltpu.sync_copy(rowA, acc_sh.at[dstA], add=True)

        plsc.subcore_barrier()
        pltpu.sync_copy(acc_sh.at[rs], acc_hbm.at[c, r].at[rs])
        plsc.subcore_barrier()


def _final_body(acc_ref, t_ref, dinv_ref, b_ref, pos_ref, neg_ref, sum_ref):
    dv = dinv_ref[0]
    bias = b_ref[0, 0]
    p = jnp.maximum(dv * (acc_ref[0, 0] + t_ref[0, 0]) + bias, 0.0)
    n = jnp.maximum(dv * (acc_ref[1, 0] + t_ref[1, 0]) + bias, 0.0)
    pos_ref[0] = p
    neg_ref[0] = n
    part = (jnp.sum(p, axis=0) * (1.0 / N)).reshape(1, 1, DO)
    b = pl.program_id(1)

    @pl.when(b == 0)
    def _():
        sum_ref[...] = part

    @pl.when(b > 0)
    def _():
        sum_ref[...] += part


def _final(acc, t, dinv, bs3):
    return pl.pallas_call(
        _final_body,
        grid=(R, _NB),
        in_specs=[
            pl.BlockSpec((NC, 1, _BR, DO), lambda r, b: (0, r, b, 0)),
            pl.BlockSpec((NC, 1, _BR, DO), lambda r, b: (0, r, b, 0)),
            pl.BlockSpec((1, _BR, 1), lambda r, b: (r, b, 0)),
            pl.BlockSpec((1, 1, DO), lambda r, b: (r, 0, 0)),
        ],
        out_specs=[
            pl.BlockSpec((1, _BR, DO), lambda r, b: (r, b, 0)),
            pl.BlockSpec((1, _BR, DO), lambda r, b: (r, b, 0)),
            pl.BlockSpec((1, 1, DO), lambda r, b: (r, 0, 0)),
        ],
        out_shape=[
            jax.ShapeDtypeStruct((R, N, DO), jnp.float32),
            jax.ShapeDtypeStruct((R, N, DO), jnp.float32),
            jax.ShapeDtypeStruct((R, 1, DO), jnp.float32),
        ],
    )(acc, t, dinv, bs3)


def kernel(x, edge_indices, Ws, bs):
    src = edge_indices[:, 0, :].astype(jnp.int32)
    dst = edge_indices[:, 1, :].astype(jnp.int32)
    perm = jax.random.permutation(jax.random.key(1), N)
    perm2d = perm.astype(jnp.int32).reshape(NS, _RPS)
    z1 = jnp.zeros((N,), jnp.float32)
    z2 = jnp.zeros((N, DO), jnp.float32)

    degp, xperm = _deg_perm(dst, x, perm2d, z1)
    t, dinv = _tables(x, xperm, Ws, degp.reshape(NC, R, N, 1))
    acc = _edge(src, dst, t, z2)
    pos, neg, summ = _final(acc, t, dinv, bs.reshape(R, 1, DO))
    return pos, neg, summ


# speedup vs baseline: 1.4772x; 1.3015x over previous
import jax
import jax.numpy as jnp
import numpy as np
from jax import lax
from jax.experimental import pallas as pl
from jax.experimental.pallas import tpu as pltpu
from jax.experimental.pallas import tpu_sc as plsc

N = 10000
DI = 128
DO = 64
R = 3
E = 320000
NC = 2
NS = 16

_ST = [s * 632 for s in range(NS - 1)] + [632 * (NS - 1)]
_LN = [632] * (NS - 1) + [N - 632 * (NS - 1)]

_RPS = N // NS
_EW_A = E // (NC * NS)
_KA = 2000
_EW_C = E // NS
_KC = 400
_NCH = _EW_C // _KC
_NPAIR = (_NCH - 2) // 2

_BR = 1000
_NB = N // _BR

_PERM_NP = np.asarray(
    jax.random.permutation(jax.random.key(1), N)
).astype(np.int32).reshape(NS, _RPS)


def _vec_mesh():
    return plsc.VectorSubcoreMesh(core_axis_name="core", subcore_axis_name="subcore")


_SC_PARAMS = pltpu.CompilerParams(use_tc_tiling_on_sc=False)


def _deg_perm(dst, x, perm2d, z1):
    kfn = pl.kernel(
        _deg_perm_body,
        out_type=(
            jax.ShapeDtypeStruct((NC, R, N), jnp.float32),
            jax.ShapeDtypeStruct((N, DI), jnp.float32),
        ),
        mesh=_vec_mesh(),
        scratch_types=[
            pltpu.VMEM_SHARED((N,), jnp.float32),
            pltpu.VMEM_SHARED((N,), jnp.float32),
            pltpu.VMEM_SHARED((N,), jnp.float32),
            pltpu.VMEM((_KA,), jnp.float32),
            pltpu.VMEM((_KA,), jnp.int32),
            pltpu.VMEM((_RPS,), jnp.int32),
            pltpu.VMEM((_RPS, DI), jnp.float32),
        ],
        compiler_params=_SC_PARAMS,
    )
    return kfn(dst, x, perm2d, z1)


def _deg_perm_body(dst_hbm, x_hbm, perm_hbm, z1_hbm, degp_hbm, xperm_hbm,
                   d0_sh, d1_sh, d2_sh, ones_v, idx_v, perm_v, xrow_v):
    c = lax.axis_index("core")
    s = lax.axis_index("subcore")
    degs = [d0_sh, d1_sh, d2_sh]

    @pl.loop(0, _KA, step=16)
    def _(i):
        ones_v[pl.ds(i, 16)] = jnp.full((16,), 1.0, jnp.float32)

    for i in range(NS):
        @pl.when(s == i)
        def _():
            for r in range(R):
                pltpu.sync_copy(z1_hbm.at[pl.ds(_ST[i], _LN[i])],
                                degs[r].at[pl.ds(_ST[i], _LN[i])])
    plsc.subcore_barrier()

    @pl.when(c == 0)
    def _():
        pltpu.sync_copy(perm_hbm.at[s], perm_v)
        pltpu.sync_copy(x_hbm.at[perm_v], xrow_v)
        pltpu.sync_copy(xrow_v, xperm_hbm.at[pl.ds(s * _RPS, _RPS)])

    base0 = (c * NS + s) * _EW_A
    for r in range(R):
        @pl.loop(0, _EW_A, step=_KA)
        def _(j):
            pltpu.sync_copy(dst_hbm.at[r, pl.ds(base0 + j, _KA)], idx_v)
            pltpu.sync_copy(ones_v, degs[r].at[idx_v], add=True)
    plsc.subcore_barrier()

    for i in range(NS):
        @pl.when(s == i)
        def _():
            for r in range(R):
                pltpu.sync_copy(degs[r].at[pl.ds(_ST[i], _LN[i])],
                                degp_hbm.at[c, r, pl.ds(_ST[i], _LN[i])])


def _tables_body(x_ref, xp_ref, w_ref, degp_ref, t_ref, dinv_ref):
    deg = degp_ref[0, 0, :, 0] + degp_ref[1, 0, :, 0] + 1.0
    dv = lax.rsqrt(deg).reshape(_BR, 1)
    w = w_ref[0]
    xw = jnp.dot(x_ref[...], w, preferred_element_type=jnp.float32,
                 precision=lax.Precision.HIGHEST)
    xwp = jnp.dot(xp_ref[...], w, preferred_element_type=jnp.float32,
                  precision=lax.Precision.HIGHEST)
    t_ref[0, 0] = dv * xw
    t_ref[1, 0] = dv * xwp
    dinv_ref[0] = dv


def _tables(x, xperm, Ws, degp):
    return pl.pallas_call(
        _tables_body,
        grid=(R, _NB),
        in_specs=[
            pl.BlockSpec((_BR, DI), lambda r, b: (b, 0)),
            pl.BlockSpec((_BR, DI), lambda r, b: (b, 0)),
            pl.BlockSpec((1, DI, DO), lambda r, b: (r, 0, 0)),
            pl.BlockSpec((NC, 1, _BR, 1), lambda r, b: (0, r, b, 0)),
        ],
        out_specs=[
            pl.BlockSpec((NC, 1, _BR, DO), lambda r, b: (0, r, b, 0)),
            pl.BlockSpec((1, _BR, 1), lambda r, b: (r, b, 0)),
        ],
        out_shape=[
            jax.ShapeDtypeStruct((NC, R, N, DO), jnp.float32),
            jax.ShapeDtypeStruct((R, N, 1), jnp.float32),
        ],
    )(x, xperm, Ws, degp)


def _edge(src, dst, t, z2):
    kfn = pl.kernel(
        _edge_body,
        out_type=jax.ShapeDtypeStruct((NC, R, N, DO), jnp.float32),
        mesh=_vec_mesh(),
        scratch_types=[
            pltpu.VMEM_SHARED((N, DO), jnp.float32),
            pltpu.VMEM((_KC,), jnp.int32),
            pltpu.VMEM((_KC,), jnp.int32),
            pltpu.VMEM((_KC,), jnp.int32),
            pltpu.VMEM((_KC,), jnp.int32),
            pltpu.VMEM((_KC, DO), jnp.float32),
            pltpu.VMEM((_KC, DO), jnp.float32),
            pltpu.SemaphoreType.DMA,
            pltpu.SemaphoreType.DMA,
        ],
        compiler_params=_SC_PARAMS,
    )
    return kfn(src, dst, t, z2)


def _edge_body(src_hbm, dst_hbm, t_hbm, z2_hbm, acc_hbm,
               acc_sh, srcA, srcB, dstA, dstB, rowA, rowB, semA, semB):
    c = lax.axis_index("core")
    s = lax.axis_index("subcore")
    rs = pl.ds(s * _RPS, _RPS)
    base0 = s * _EW_C
    for r in range(R):
        pltpu.sync_copy(z2_hbm.at[rs], acc_sh.at[rs])
        plsc.subcore_barrier()

        tcr = t_hbm.at[c, r]
        pltpu.sync_copy(src_hbm.at[r, pl.ds(base0, _KC)], srcA)
        pltpu.sync_copy(dst_hbm.at[r, pl.ds(base0, _KC)], dstA)
        pltpu.async_copy(tcr.at[srcA], rowA, semA)

        @pl.loop(0, _NPAIR * 2 * _KC, step=2 * _KC)
        def _(j):
            pltpu.sync_copy(src_hbm.at[r, pl.ds(base0 + j + _KC, _KC)], srcB)
            pltpu.sync_copy(dst_hbm.at[r, pl.ds(base0 + j + _KC, _KC)], dstB)
            pltpu.async_copy(tcr.at[srcB], rowB, semB)
            pltpu.make_async_copy(tcr.at[srcA], rowA, semA).wait()
            pltpu.sync_copy(rowA, acc_sh.at[dstA], add=True)

            nj = j + 2 * _KC
            pltpu.sync_copy(src_hbm.at[r, pl.ds(base0 + nj, _KC)], srcA)
            pltpu.sync_copy(dst_hbm.at[r, pl.ds(base0 + nj, _KC)], dstA)
            pltpu.async_copy(tcr.at[srcA], rowA, semA)
            pltpu.make_async_copy(tcr.at[srcB], rowB, semB).wait()
            pltpu.sync_copy(rowB, acc_sh.at[dstB], add=True)

        pltpu.make_async_copy(tcr.at[srcA], rowA, semA).wait()
        pltpu.sync_copy(rowA, acc_sh.at[dstA], add=True)
        last = base0 + (_NCH - 1) * _KC
        pltpu.sync_copy(src_hbm.at[r, pl.ds(last, _KC)], srcB)
        pltpu.sync_copy(dst_hbm.at[r, pl.ds(last, _KC)], dstB)
        pltpu.sync_copy(tcr.at[srcB], rowB)
        pltpu.sync_copy(rowB, acc_sh.at[dstB], add=True)

        plsc.subcore_barrier()
        pltpu.sync_copy(acc_sh.at[rs], acc_hbm.at[c, r].at[rs])
        plsc.subcore_barrier()


def _final_body(acc_ref, t_ref, dinv_ref, b_ref, pos_ref, neg_ref, sum_ref):
    dv = dinv_ref[0]
    bias = b_ref[0, 0]
    p = jnp.maximum(dv * (acc_ref[0, 0] + t_ref[0, 0]) + bias, 0.0)
    n = jnp.maximum(dv * (acc_ref[1, 0] + t_ref[1, 0]) + bias, 0.0)
    pos_ref[0] = p
    neg_ref[0] = n
    part = (jnp.sum(p, axis=0) * (1.0 / N)).reshape(1, 1, DO)
    b = pl.program_id(1)

    @pl.when(b == 0)
    def _():
        sum_ref[...] = part

    @pl.when(b > 0)
    def _():
        sum_ref[...] += part


def _final(acc, t, dinv, bs3):
    return pl.pallas_call(
        _final_body,
        grid=(R, _NB),
        in_specs=[
            pl.BlockSpec((NC, 1, _BR, DO), lambda r, b: (0, r, b, 0)),
            pl.BlockSpec((NC, 1, _BR, DO), lambda r, b: (0, r, b, 0)),
            pl.BlockSpec((1, _BR, 1), lambda r, b: (r, b, 0)),
            pl.BlockSpec((1, 1, DO), lambda r, b: (r, 0, 0)),
        ],
        out_specs=[
            pl.BlockSpec((1, _BR, DO), lambda r, b: (r, b, 0)),
            pl.BlockSpec((1, _BR, DO), lambda r, b: (r, b, 0)),
            pl.BlockSpec((1, 1, DO), lambda r, b: (r, 0, 0)),
        ],
        out_shape=[
            jax.ShapeDtypeStruct((R, N, DO), jnp.float32),
            jax.ShapeDtypeStruct((R, N, DO), jnp.float32),
            jax.ShapeDtypeStruct((R, 1, DO), jnp.float32),
        ],
    )(acc, t, dinv, bs3)


def kernel(x, edge_indices, Ws, bs):
    src = edge_indices[:, 0, :].astype(jnp.int32)
    dst = edge_indices[:, 1, :].astype(jnp.int32)
    perm2d = jnp.asarray(_PERM_NP)
    z1 = jnp.zeros((N,), jnp.float32)
    z2 = jnp.zeros((N, DO), jnp.float32)

    degp, xperm = _deg_perm(dst, x, perm2d, z1)
    t, dinv = _tables(x, xperm, Ws, degp.reshape(NC, R, N, 1))
    acc = _edge(src, dst, t, z2)
    pos, neg, summ = _final(acc, t, dinv, bs.reshape(R, 1, DO))
    return pos, neg, summ
